# baseline (speedup 1.0000x reference)
import functools
import jax
import jax.numpy as jnp
from jax import lax
from jax.experimental import pallas as pl
from jax.experimental.pallas import tpu as pltpu
from jax.experimental.pallas import tpu_sc as plsc

_NC, _NS = 2, 16
_NW = _NC * _NS
_K = 4


def _make_sc_copy(rows, d, dtype):
    per = rows // _NW
    ch = per // _K
    mesh = plsc.VectorSubcoreMesh(
        core_axis_name="c", subcore_axis_name="s",
        num_cores=_NC, num_subcores=_NS,
    )

    @functools.partial(
        pl.kernel,
        out_type=jax.ShapeDtypeStruct((rows, d), dtype),
        mesh=mesh,
        scratch_types=[
            pltpu.VMEM((per, d), dtype),
            pltpu.SemaphoreType.DMA,
            pltpu.SemaphoreType.DMA,
        ],
    )
    def sc_copy(pe_hbm, out_hbm, buf, rsem, wsem):
        wid = lax.axis_index("s") * _NC + lax.axis_index("c")
        base = wid * per
        reads = []
        for c in range(_K):
            cp = pltpu.make_async_copy(
                pe_hbm.at[pl.ds(base + c * ch, ch)],
                buf.at[pl.ds(c * ch, ch)],
                rsem,
            )
            cp.start()
            reads.append(cp)
        writes = []
        for c in range(_K):
            reads[c].wait()
            wp = pltpu.make_async_copy(
                buf.at[pl.ds(c * ch, ch)],
                out_hbm.at[pl.ds(base + c * ch, ch)],
                wsem,
            )
            wp.start()
            writes.append(wp)
        for wp in writes:
            wp.wait()

    return sc_copy


def kernel(x, pe):
    seq_len = x.shape[1]
    d = pe.shape[2]
    pe2 = pe.reshape(pe.shape[1], d)
    out = _make_sc_copy(seq_len, d, pe.dtype)(pe2)
    return out.reshape(1, seq_len, d)


# speedup vs baseline: 4.1684x; 4.1684x over previous
import jax
import jax.numpy as jnp
from jax.experimental import pallas as pl
from jax.experimental.pallas import tpu as pltpu

_N_CH = 8


def _copy_body(pe_hbm, out_hbm, buf, rsem, wsem):
    rows = out_hbm.shape[0]
    ch = rows // _N_CH
    reads = []
    for c in range(_N_CH):
        cp = pltpu.make_async_copy(
            pe_hbm.at[pl.ds(c * ch, ch)],
            buf.at[pl.ds(c * ch, ch)],
            rsem,
        )
        cp.start()
        reads.append(cp)
    writes = []
    for c in range(_N_CH):
        reads[c].wait()
        wp = pltpu.make_async_copy(
            buf.at[pl.ds(c * ch, ch)],
            out_hbm.at[pl.ds(c * ch, ch)],
            wsem,
        )
        wp.start()
        writes.append(wp)
    for wp in writes:
        wp.wait()


def kernel(x, pe):
    seq_len = x.shape[1]
    d = pe.shape[2]
    pe2 = pe.reshape(pe.shape[1], d)
    out = pl.pallas_call(
        _copy_body,
        in_specs=[pl.BlockSpec(memory_space=pltpu.MemorySpace.HBM)],
        out_specs=pl.BlockSpec(memory_space=pltpu.MemorySpace.HBM),
        out_shape=jax.ShapeDtypeStruct((seq_len, d), pe.dtype),
        scratch_shapes=[
            pltpu.VMEM((seq_len, d), pe.dtype),
            pltpu.SemaphoreType.DMA,
            pltpu.SemaphoreType.DMA,
        ],
    )(pe2)
    return out.reshape(1, seq_len, d)
